# baseline (speedup 1.0000x reference)
import functools

import jax
import jax.numpy as jnp
from jax import lax
from jax.experimental import pallas as pl
from jax.experimental.pallas import tpu as pltpu
from jax.experimental.pallas import tpu_sc as plsc

NUM_ENTITIES = 100000
X_DIM = 16
N_TABLES = 8
BATCH = 16384

NC = 2
NS = 16
NW = NC * NS
N_ROWS = N_TABLES * X_DIM
ROWS_PER_W = N_ROWS // NW
FB = X_DIM // 8
L = 16
BB = BATCH // 128
HALF_BB = BB // 2

OUT_BLOCK_BYTES = 128 * 4


@functools.partial(
    pl.kernel,
    mesh=plsc.VectorSubcoreMesh(core_axis_name="c", subcore_axis_name="s"),
    out_type=jax.ShapeDtypeStruct(
        (N_TABLES, FB, NW, 4, 8, 128), jnp.float32
    ),
    scratch_types=[
        pltpu.VMEM((BATCH,), jnp.int32),
        pltpu.VMEM((NUM_ENTITIES,), jnp.float32),
        pltpu.VMEM((HALF_BB * 128,), jnp.float32),
        pltpu.SemaphoreType.DMA,
        pltpu.SemaphoreType.DMA,
    ],
    compiler_params=pltpu.CompilerParams(
        use_tc_tiling_on_sc=True, needs_layout_passes=False
    ),
)
def _stacked_gather(x_hbm, tab_hbm, out_hbm, idx_v, row_v, ostage_v, wsem, rsem):
    wid = lax.axis_index("s") * NC + lax.axis_index("c")
    idx_copy = pltpu.async_copy(x_hbm, idx_v, rsem)

    def drain_half():
        pltpu.make_async_copy(
            x_hbm.at[pl.ds(0, HALF_BB * 128)],
            idx_v.at[pl.ds(0, HALF_BB * 128)],
            wsem,
        ).wait()

    for k in range(ROWS_PER_W):
        r = wid * ROWS_PER_W + k
        t = r // X_DIM
        f = r % X_DIM
        fb = f // 8
        f_in = f % 8
        row_copy = pltpu.async_copy(tab_hbm.at[t, f], row_v, rsem)
        if k == 0:
            idx_copy.wait()
        row_copy.wait()
        for half in range(2):
            if k > 0 or half > 0:
                drain_half()

            def bb_body(bb, _):
                bb_g = half * HALF_BB + bb
                b0 = bb_g * 128
                for j in range(8):
                    ivec = idx_v[pl.ds(b0 + j * L, L)]
                    vec = plsc.load_gather(row_v, [ivec])
                    ostage_v[pl.ds(bb * 128 + j * L, L)] = vec
                pltpu.async_copy(
                    ostage_v.at[pl.ds(bb * 128, 128)],
                    out_hbm.at[t, fb, bb_g // 4, bb_g % 4, f_in],
                    wsem,
                )
                return None

            lax.fori_loop(0, HALF_BB, bb_body, None, unroll=8)
    drain_half()


def kernel(x, ent_tables):
    tt = ent_tables.transpose(0, 2, 1)
    raw = _stacked_gather(x, tt)
    return raw.transpose(0, 2, 3, 5, 1, 4).reshape(N_TABLES, BATCH, X_DIM)


# speedup vs baseline: 1.0268x; 1.0268x over previous
import functools

import jax
import jax.numpy as jnp
from jax import lax
from jax.experimental import pallas as pl
from jax.experimental.pallas import tpu as pltpu
from jax.experimental.pallas import tpu_sc as plsc

NUM_ENTITIES = 100000
X_DIM = 16
N_TABLES = 8
BATCH = 16384

NC = 2
NS = 16
NW = NC * NS
N_ROWS = N_TABLES * X_DIM
ROWS_PER_W = N_ROWS // NW
FB = X_DIM // 8
L = 16
BB = BATCH // 128
HALF_BB = BB // 2

OUT_BLOCK_BYTES = 128 * 4


@functools.partial(
    pl.kernel,
    mesh=plsc.VectorSubcoreMesh(core_axis_name="c", subcore_axis_name="s"),
    out_type=jax.ShapeDtypeStruct(
        (N_TABLES, FB, NW, 4, 8, 128), jnp.float32
    ),
    scratch_types=[
        pltpu.VMEM((BATCH,), jnp.int32),
        pltpu.VMEM((NUM_ENTITIES,), jnp.float32),
        pltpu.VMEM((HALF_BB * 128,), jnp.float32),
        pltpu.SemaphoreType.DMA,
        pltpu.SemaphoreType.DMA,
    ],
    compiler_params=pltpu.CompilerParams(
        use_tc_tiling_on_sc=True, needs_layout_passes=False
    ),
)
def _stacked_gather(x_hbm, tab_hbm, out_hbm, idx_v, row_v, ostage_v, wsem, rsem):
    wid = lax.axis_index("s") * NC + lax.axis_index("c")
    idx_copy = pltpu.async_copy(x_hbm, idx_v, rsem)

    def drain_half():
        pltpu.make_async_copy(
            x_hbm.at[pl.ds(0, HALF_BB * 128)],
            idx_v.at[pl.ds(0, HALF_BB * 128)],
            wsem,
        ).wait()

    for k in range(ROWS_PER_W):
        r = wid * ROWS_PER_W + k
        t = r // X_DIM
        f = r % X_DIM
        fb = f // 8
        f_in = f % 8
        row_copy = pltpu.async_copy(tab_hbm.at[t, f], row_v, rsem)
        if k == 0:
            idx_copy.wait()
        row_copy.wait()
        for half in range(2):
            if k > 0 or half > 0:
                drain_half()

            def bb_body(bb, _):
                bb_g = half * HALF_BB + bb
                b0 = bb_g * 128
                for j in range(8):
                    ivec = idx_v[pl.ds(b0 + j * L, L)]
                    vec = plsc.load_gather(row_v, [ivec])
                    ostage_v[pl.ds(bb * 128 + j * L, L)] = vec
                pltpu.async_copy(
                    ostage_v.at[pl.ds(bb * 128, 128)],
                    out_hbm.at[t, fb, bb_g // 4, bb_g % 4, f_in],
                    wsem,
                )
                return None

            lax.fori_loop(0, HALF_BB, bb_body, None, unroll=4)
    drain_half()


def kernel(x, ent_tables):
    tt = ent_tables.transpose(0, 2, 1)
    raw = _stacked_gather(x, tt)
    return raw.transpose(0, 2, 3, 5, 1, 4).reshape(N_TABLES, BATCH, X_DIM)


# speedup vs baseline: 1.0270x; 1.0002x over previous
import functools

import jax
import jax.numpy as jnp
from jax import lax
from jax.experimental import pallas as pl
from jax.experimental.pallas import tpu as pltpu
from jax.experimental.pallas import tpu_sc as plsc

NUM_ENTITIES = 100000
X_DIM = 16
N_TABLES = 8
BATCH = 16384

NC = 2
NS = 16
NW = NC * NS
N_ROWS = N_TABLES * X_DIM
ROWS_PER_W = N_ROWS // NW
FB = X_DIM // 8
L = 16
BB = BATCH // 128
HALF_BB = BB // 2


@functools.partial(
    pl.kernel,
    mesh=plsc.VectorSubcoreMesh(core_axis_name="c", subcore_axis_name="s"),
    out_type=jax.ShapeDtypeStruct(
        (N_TABLES, FB, NW, 4, 8, 128), jnp.float32
    ),
    scratch_types=[
        pltpu.VMEM((BATCH,), jnp.int32),
        pltpu.VMEM((NUM_ENTITIES,), jnp.float32),
        pltpu.VMEM((HALF_BB * 128,), jnp.float32),
        pltpu.SemaphoreType.DMA,
        pltpu.SemaphoreType.DMA,
    ],
    compiler_params=pltpu.CompilerParams(
        use_tc_tiling_on_sc=True, needs_layout_passes=False
    ),
)
def _stacked_gather(x_hbm, tab_hbm, out_hbm, idx_v, row_v, ostage_v, wsem, rsem):
    wid = lax.axis_index("s") * NC + lax.axis_index("c")
    idx_copy = pltpu.async_copy(x_hbm, idx_v, rsem)

    def drain_half():
        pltpu.make_async_copy(
            x_hbm.at[pl.ds(0, HALF_BB * 128)],
            idx_v.at[pl.ds(0, HALF_BB * 128)],
            wsem,
        ).wait()

    for k in range(ROWS_PER_W):
        r = wid * ROWS_PER_W + k
        t = r // X_DIM
        f = r % X_DIM
        fb = f // 8
        f_in = f % 8
        row_copy = pltpu.async_copy(tab_hbm.at[t, f], row_v, rsem)
        if k == 0:
            idx_copy.wait()
        row_copy.wait()
        for half in range(2):
            if k > 0 or half > 0:
                drain_half()

            def bb_body(bb, _):
                bb_g = half * HALF_BB + bb
                b0 = bb_g * 128
                for j in range(8):
                    ivec = idx_v[pl.ds(b0 + j * L, L)]
                    vec = plsc.load_gather(row_v, [ivec])
                    ostage_v[pl.ds(bb * 128 + j * L, L)] = vec
                pltpu.async_copy(
                    ostage_v.at[pl.ds(bb * 128, 128)],
                    out_hbm.at[t, fb, bb_g // 4, bb_g % 4, f_in],
                    wsem,
                )
                return None

            lax.fori_loop(0, HALF_BB, bb_body, None, unroll=4)
    drain_half()


def kernel(x, ent_tables):
    tt = ent_tables.transpose(0, 2, 1)
    raw = _stacked_gather(x, tt)
    return raw.transpose(0, 2, 3, 5, 1, 4).reshape(N_TABLES, BATCH, X_DIM)
